# baseline (speedup 1.0000x reference)
import dataclasses

import jax
import jax.numpy as jnp
from jax import lax
from jax.experimental import pallas as pl
from jax.experimental.pallas import tpu as pltpu
from jax.experimental.pallas import tpu_sc as plsc

N = 10000
E = 320000
H = 128
HH = H // 2
G = 64
NS = 16
EPT = E // NS
K = 80
NCH = EPT // K
NST = 5
SCH = NCH // NST
AGR = 10240
RPT = AGR // NS
ZB = RPT // K
NB = 5
BR = N // NB
F4 = HH // 16


def _fold_body(wn, bn, we, be, we1, be1, we2, be2, p_ref):
    v1 = jnp.dot(we[...], we1[...], preferred_element_type=jnp.float32)
    c1 = jnp.dot(be[...], we1[...], preferred_element_type=jnp.float32) + be1[...]
    v2 = jnp.dot(we[...], we2[...], preferred_element_type=jnp.float32)
    c2 = jnp.dot(be[...], we2[...], preferred_element_type=jnp.float32) + be2[...]
    z = jnp.zeros((2, H), jnp.float32)
    p_ref[...] = jnp.concatenate(
        [wn[...], v1, c1 + bn[...], bn[...], v2, c2, z], axis=0)


def _fold(wn, bn, we, be, we1, be1, we2, be2):
    return pl.pallas_call(
        _fold_body,
        out_shape=jax.ShapeDtypeStruct((8, H), jnp.float32),
    )(wn, bn, we, be, we1, be1, we2, be2)


_SC_MESH = plsc.VectorSubcoreMesh(core_axis_name="c", subcore_axis_name="s")
_SC_PARAMS = pltpu.CompilerParams()
if "needs_layout_passes" in pltpu.CompilerParams.__dataclass_fields__:
    _SC_PARAMS = dataclasses.replace(_SC_PARAMS, needs_layout_passes=False)
if "use_tc_tiling_on_sc" in pltpu.CompilerParams.__dataclass_fields__:
    _SC_PARAMS = dataclasses.replace(_SC_PARAMS, use_tc_tiling_on_sc=False)


def _zero_init(s, msg_v, aggr_sh):
    def zrow(r):
        for f in range(F4):
            msg_v[r, pl.ds(16 * f, 16)] = jnp.zeros((16,), jnp.float32)
    pl.loop(0, K)(zrow)

    def blk(i):
        pltpu.sync_copy(msg_v, aggr_sh.at[pl.ds(s * RPT + i * K, K), :])
    pl.loop(0, ZB)(blk)


def _copy_out(c, s, aggr_sh, out0, out1):
    @pl.when(c == 0)
    def _():
        pltpu.sync_copy(aggr_sh.at[pl.ds(s * RPT, RPT), :],
                        out0.at[pl.ds(s * RPT, RPT), :])

    @pl.when(c == 1)
    def _():
        pltpu.sync_copy(aggr_sh.at[pl.ds(s * RPT, RPT), :],
                        out1.at[pl.ds(s * RPT, RPT), :])


def _sc1_body(x_hbm, src_hbm, dst_hbm, ea_hbm, p_hbm,
              out0, out1,
              x_v, src_v, dst_v, ea_v, xs_v, msg0_v, msg1_v, w_v,
              aggr_sh, sem0, sem1):
    c = lax.axis_index("c")
    s = lax.axis_index("s")
    pltpu.sync_copy(x_hbm, x_v)
    pltpu.sync_copy(p_hbm, w_v)

    cb = c * HH
    wn = [w_v[0, pl.ds(cb + 16 * f, 16)] for f in range(F4)]
    v1 = [w_v[1, pl.ds(cb + 16 * f, 16)] for f in range(F4)]
    d1 = [w_v[2, pl.ds(cb + 16 * f, 16)] for f in range(F4)]

    _zero_init(s, msg0_v, aggr_sh)
    plsc.subcore_barrier()

    def compute_chunk(ch, mv):
        chv = jnp.full((16,), ch, jnp.int32)
        for j in range(K // 16):
            idx16 = src_v[ch, pl.ds(j * 16, 16)]
            xs_v[pl.ds(j * 16, 16)] = plsc.load_gather(x_v, [idx16])

        def edge(jj):
            jv = jnp.full((16,), jj, jnp.int32)
            xj = plsc.load_gather(xs_v, [jv])
            tj = plsc.load_gather(ea_v, [chv, jv])
            for f in range(F4):
                mv[jj, pl.ds(16 * f, 16)] = jnp.maximum(
                    xj * wn[f] + tj * v1[f] + d1[f], 0.0)
        plsc.parallel_loop(0, K, unroll=16)(edge)

    bufs = ((msg0_v, sem0), (msg1_v, sem1))

    def stage(t):
        pltpu.sync_copy(src_hbm.at[s, t], src_v)
        pltpu.sync_copy(dst_hbm.at[s, t], dst_v)
        pltpu.sync_copy(ea_hbm.at[s, t], ea_v)
        for ch in range(2):
            mv, sem = bufs[ch]
            compute_chunk(ch, mv)
            pltpu.async_copy(mv, aggr_sh.at[dst_v.at[ch]], sem, add=True)

        def body(chb):
            for b in range(2):
                ch = chb + b
                mv, sem = bufs[b]
                pltpu.make_async_copy(
                    mv, aggr_sh.at[dst_v.at[ch]], sem).wait()
                compute_chunk(ch, mv)
                pltpu.async_copy(mv, aggr_sh.at[dst_v.at[ch]], sem, add=True)
        pl.loop(2, SCH, step=2)(body)
        for b in range(2):
            mv, sem = bufs[b]
            pltpu.make_async_copy(mv, aggr_sh.at[dst_v.at[b]], sem).wait()
    pl.loop(0, NST)(stage)

    plsc.subcore_barrier()
    _copy_out(c, s, aggr_sh, out0, out1)


def _sc_layer1(x, src3, dst3, ea3, p):
    kern = pl.kernel(
        _sc1_body,
        out_type=[pltpu.HBM((AGR, HH), jnp.float32),
                  pltpu.HBM((AGR, HH), jnp.float32)],
        mesh=_SC_MESH,
        compiler_params=_SC_PARAMS,
        scratch_types=[
            pltpu.VMEM((N,), jnp.float32),
            pltpu.VMEM((SCH, K), jnp.int32),
            pltpu.VMEM((SCH, K), jnp.int32),
            pltpu.VMEM((SCH, K), jnp.float32),
            pltpu.VMEM((K,), jnp.float32),
            pltpu.VMEM((K, HH), jnp.float32),
            pltpu.VMEM((K, HH), jnp.float32),
            pltpu.VMEM((8, H), jnp.float32),
            pltpu.VMEM_SHARED((AGR, HH), jnp.float32),
            pltpu.SemaphoreType.DMA,
            pltpu.SemaphoreType.DMA,
        ],
    )
    return kern(x, src3, dst3, ea3, p)


def _sc2_body(ha_hbm, hb_hbm, src_hbm, dst_hbm, ea_hbm, p_hbm,
              out0, out1,
              src_v, dst_v, ea_v, rows0_v, rows1_v, msg0_v, msg1_v, w_v,
              aggr_sh, gsem0, gsem1, ssem0, ssem1):
    c = lax.axis_index("c")
    s = lax.axis_index("s")
    pltpu.sync_copy(p_hbm, w_v)

    cb = c * HH
    v2 = [w_v[4, pl.ds(cb + 16 * f, 16)] for f in range(F4)]
    d2 = [w_v[5, pl.ds(cb + 16 * f, 16)] for f in range(F4)]

    _zero_init(s, msg0_v, aggr_sh)
    plsc.subcore_barrier()

    def issue_gather(ch, rv, gsem):
        @pl.when(c == 0)
        def _():
            pltpu.async_copy(ha_hbm.at[src_v.at[ch]], rv, gsem)

        @pl.when(c == 1)
        def _():
            pltpu.async_copy(hb_hbm.at[src_v.at[ch]], rv, gsem)

    def wait_gather(ch, rv, gsem):
        @pl.when(c == 0)
        def _():
            pltpu.make_async_copy(ha_hbm.at[src_v.at[ch]], rv, gsem).wait()

        @pl.when(c == 1)
        def _():
            pltpu.make_async_copy(hb_hbm.at[src_v.at[ch]], rv, gsem).wait()

    def compute_chunk(ch, rv, mv):
        chv = jnp.full((16,), ch, jnp.int32)

        def edge(jj):
            jv = jnp.full((16,), jj, jnp.int32)
            tj = plsc.load_gather(ea_v, [chv, jv])
            for f in range(F4):
                sl = pl.ds(16 * f, 16)
                mv[jj, sl] = jnp.maximum(
                    rv[jj, sl] + tj * v2[f] + d2[f], 0.0)
        plsc.parallel_loop(0, K, unroll=16)(edge)

    bufs = ((rows0_v, msg0_v, gsem0, ssem0), (rows1_v, msg1_v, gsem1, ssem1))

    def stage(t):
        pltpu.sync_copy(src_hbm.at[s, t], src_v)
        pltpu.sync_copy(dst_hbm.at[s, t], dst_v)
        pltpu.sync_copy(ea_hbm.at[s, t], ea_v)
        issue_gather(0, rows0_v, gsem0)
        issue_gather(1, rows1_v, gsem1)
        for ch in range(2):
            rv, mv, gsem, ssem = bufs[ch]
            wait_gather(ch, rv, gsem)
            compute_chunk(ch, rv, mv)
            issue_gather(ch + 2, rv, gsem)
            pltpu.async_copy(mv, aggr_sh.at[dst_v.at[ch]], ssem, add=True)

        def body(chb):
            for b in range(2):
                ch = chb + b
                rv, mv, gsem, ssem = bufs[b]
                wait_gather(ch, rv, gsem)
                pltpu.make_async_copy(
                    mv, aggr_sh.at[dst_v.at[ch]], ssem).wait()
                compute_chunk(ch, rv, mv)

                @pl.when(ch + 2 < SCH)
                def _():
                    issue_gather(ch + 2, rv, gsem)
                pltpu.async_copy(mv, aggr_sh.at[dst_v.at[ch]], ssem, add=True)
        pl.loop(2, SCH, step=2)(body)
        for b in range(2):
            rv, mv, gsem, ssem = bufs[b]
            pltpu.make_async_copy(mv, aggr_sh.at[dst_v.at[b]], ssem).wait()
    pl.loop(0, NST)(stage)

    plsc.subcore_barrier()
    _copy_out(c, s, aggr_sh, out0, out1)


def _sc_layer2(h1a, h1b, src3, dst3, ea3, p):
    kern = pl.kernel(
        _sc2_body,
        out_type=[pltpu.HBM((AGR, HH), jnp.float32),
                  pltpu.HBM((AGR, HH), jnp.float32)],
        mesh=_SC_MESH,
        compiler_params=_SC_PARAMS,
        scratch_types=[
            pltpu.VMEM((SCH, K), jnp.int32),
            pltpu.VMEM((SCH, K), jnp.int32),
            pltpu.VMEM((SCH, K), jnp.float32),
            pltpu.VMEM((K, HH), jnp.float32),
            pltpu.VMEM((K, HH), jnp.float32),
            pltpu.VMEM((K, HH), jnp.float32),
            pltpu.VMEM((K, HH), jnp.float32),
            pltpu.VMEM((8, H), jnp.float32),
            pltpu.VMEM_SHARED((AGR, HH), jnp.float32),
            pltpu.SemaphoreType.DMA,
            pltpu.SemaphoreType.DMA,
            pltpu.SemaphoreType.DMA,
            pltpu.SemaphoreType.DMA,
        ],
    )
    return kern(h1a, h1b, src3, dst3, ea3, p)


def _mlp_body(x2, p0, p1, pw, wa, ba, wb, bb, outa, outb):
    h0 = x2[...] * pw[0:1, :] + pw[3:4, :]
    z = h0 + jnp.concatenate([p0[...], p1[...]], axis=1)
    t = jnp.maximum(
        jnp.dot(z, wa[...], preferred_element_type=jnp.float32) + ba[...], 0.0)
    h1 = jnp.maximum(
        jnp.dot(t, wb[...], preferred_element_type=jnp.float32) + bb[...], 0.0)
    outa[...] = h1[:, :HH]
    outb[...] = h1[:, HH:]


def _mlp(x2, p0, p1, pw, wa, ba, wb, bb):
    full = lambda i: (0, 0)
    return pl.pallas_call(
        _mlp_body,
        grid=(NB,),
        in_specs=[
            pl.BlockSpec((BR, 1), lambda i: (i, 0)),
            pl.BlockSpec((BR, HH), lambda i: (i, 0)),
            pl.BlockSpec((BR, HH), lambda i: (i, 0)),
            pl.BlockSpec((8, H), full),
            pl.BlockSpec((H, H), full),
            pl.BlockSpec((1, H), full),
            pl.BlockSpec((H, H), full),
            pl.BlockSpec((1, H), full),
        ],
        out_specs=[pl.BlockSpec((BR, HH), lambda i: (i, 0)),
                   pl.BlockSpec((BR, HH), lambda i: (i, 0))],
        out_shape=[jax.ShapeDtypeStruct((N, HH), jnp.float32),
                   jax.ShapeDtypeStruct((N, HH), jnp.float32)],
    )(x2, p0, p1, pw, wa, ba, wb, bb)


def _mlp_pool_body(ha, hb, q0, q1, batch, wa, ba, wb, bb, out, sums, cnts):
    i = pl.program_id(0)

    @pl.when(i == 0)
    def _():
        sums[...] = jnp.zeros((G, H), jnp.float32)
        cnts[...] = jnp.zeros((G, H), jnp.float32)

    z = jnp.concatenate([ha[...] + q0[...], hb[...] + q1[...]], axis=1)
    t = jnp.maximum(
        jnp.dot(z, wa[...], preferred_element_type=jnp.float32) + ba[...], 0.0)
    h2 = jnp.maximum(
        jnp.dot(t, wb[...], preferred_element_type=jnp.float32) + bb[...], 0.0)
    brow = batch[0, 0, :]
    oh = (lax.broadcasted_iota(jnp.int32, (G, BR), 0)
          == brow[None, :]).astype(jnp.float32)
    sums[...] += jnp.dot(oh, h2, preferred_element_type=jnp.float32)
    cnts[...] += jnp.dot(oh, jnp.ones((BR, H), jnp.float32),
                         preferred_element_type=jnp.float32)

    @pl.when(i == NB - 1)
    def _():
        out[...] = sums[...] / jnp.maximum(cnts[...], 1.0)


def _mlp_pool(ha, hb, q0, q1, batch3, wa, ba, wb, bb):
    full = lambda i: (0, 0)
    return pl.pallas_call(
        _mlp_pool_body,
        grid=(NB,),
        in_specs=[
            pl.BlockSpec((BR, HH), lambda i: (i, 0)),
            pl.BlockSpec((BR, HH), lambda i: (i, 0)),
            pl.BlockSpec((BR, HH), lambda i: (i, 0)),
            pl.BlockSpec((BR, HH), lambda i: (i, 0)),
            pl.BlockSpec((1, 1, BR), lambda i: (i, 0, 0)),
            pl.BlockSpec((H, H), full),
            pl.BlockSpec((1, H), full),
            pl.BlockSpec((H, H), full),
            pl.BlockSpec((1, H), full),
        ],
        out_specs=pl.BlockSpec((G, H), full),
        out_shape=jax.ShapeDtypeStruct((G, H), jnp.float32),
        scratch_shapes=[
            pltpu.VMEM((G, H), jnp.float32),
            pltpu.VMEM((G, H), jnp.float32),
        ],
    )(ha, hb, q0, q1, batch3, wa, ba, wb, bb)


def kernel(x, edge_index, edge_attr, batch,
           W_node, b_node, W_edge, b_edge,
           W_e1, b_e1, W_m1a, b_m1a, W_m1b, b_m1b,
           W_e2, b_e2, W_m2a, b_m2a, W_m2b, b_m2b):
    x = x.astype(jnp.float32)
    src3 = edge_index[0].astype(jnp.int32).reshape(NS, NST, SCH, K)
    dst3 = edge_index[1].astype(jnp.int32).reshape(NS, NST, SCH, K)
    ea3 = edge_attr.astype(jnp.float32).reshape(NS, NST, SCH, K)
    batch3 = batch.astype(jnp.int32).reshape(NB, 1, BR)

    r = lambda b: b.reshape(1, H)
    p = _fold(W_node, r(b_node), W_edge, r(b_edge),
              W_e1, r(b_e1), W_e2, r(b_e2))
    p0, p1 = _sc_layer1(x, src3, dst3, ea3, p)
    h1a, h1b = _mlp(x.reshape(N, 1), p0, p1, p,
                    W_m1a, r(b_m1a), W_m1b, r(b_m1b))
    q0, q1 = _sc_layer2(h1a, h1b, src3, dst3, ea3, p)
    return _mlp_pool(h1a, h1b, q0, q1, batch3,
                     W_m2a, r(b_m2a), W_m2b, r(b_m2b))


# speedup vs baseline: 1.7279x; 1.7279x over previous
import dataclasses

import jax
import jax.numpy as jnp
from jax import lax
from jax.experimental import pallas as pl
from jax.experimental.pallas import tpu as pltpu
from jax.experimental.pallas import tpu_sc as plsc

N = 10000
E = 320000
H = 128
HH = H // 2
G = 64
NS = 16
EPT = E // NS
K = 80
NCH = EPT // K
NST = 5
SCH = NCH // NST
AGR = 10240
RPT = AGR // NS
ZB = RPT // K
NB = 5
BR = N // NB
F4 = HH // 16


def _fold_body(wn, bn, we, be, we1, be1, we2, be2, p_ref):
    v1 = jnp.dot(we[...], we1[...], preferred_element_type=jnp.float32)
    c1 = jnp.dot(be[...], we1[...], preferred_element_type=jnp.float32) + be1[...]
    v2 = jnp.dot(we[...], we2[...], preferred_element_type=jnp.float32)
    c2 = jnp.dot(be[...], we2[...], preferred_element_type=jnp.float32) + be2[...]
    z = jnp.zeros((2, H), jnp.float32)
    p_ref[...] = jnp.concatenate(
        [wn[...], v1, c1 + bn[...], bn[...], v2, c2, z], axis=0)


def _fold(wn, bn, we, be, we1, be1, we2, be2):
    return pl.pallas_call(
        _fold_body,
        out_shape=jax.ShapeDtypeStruct((8, H), jnp.float32),
    )(wn, bn, we, be, we1, be1, we2, be2)


_SC_MESH = plsc.VectorSubcoreMesh(core_axis_name="c", subcore_axis_name="s")
_SC_PARAMS = pltpu.CompilerParams()
if "needs_layout_passes" in pltpu.CompilerParams.__dataclass_fields__:
    _SC_PARAMS = dataclasses.replace(_SC_PARAMS, needs_layout_passes=False)
if "use_tc_tiling_on_sc" in pltpu.CompilerParams.__dataclass_fields__:
    _SC_PARAMS = dataclasses.replace(_SC_PARAMS, use_tc_tiling_on_sc=False)


def _zero_init(s, msg_v, aggr_sh):
    def zrow(r):
        for f in range(F4):
            msg_v[r, pl.ds(16 * f, 16)] = jnp.zeros((16,), jnp.float32)
    pl.loop(0, K)(zrow)

    def blk(i):
        pltpu.sync_copy(msg_v, aggr_sh.at[pl.ds(s * RPT + i * K, K), :])
    pl.loop(0, ZB)(blk)


def _copy_out(c, s, aggr_sh, out0, out1):
    @pl.when(c == 0)
    def _():
        pltpu.sync_copy(aggr_sh.at[pl.ds(s * RPT, RPT), :],
                        out0.at[pl.ds(s * RPT, RPT), :])

    @pl.when(c == 1)
    def _():
        pltpu.sync_copy(aggr_sh.at[pl.ds(s * RPT, RPT), :],
                        out1.at[pl.ds(s * RPT, RPT), :])


def _sc1_body(x_hbm, src_hbm, dst_hbm, ea_hbm, p_hbm,
              out0, out1,
              x_v, src_v, dst_v, ea_v, xs_v, msg0_v, msg1_v, w_v,
              aggr_sh, sem0, sem1):
    c = lax.axis_index("c")
    s = lax.axis_index("s")
    pltpu.sync_copy(x_hbm, x_v)
    pltpu.sync_copy(p_hbm, w_v)

    cb = c * HH
    wn = [w_v[0, pl.ds(cb + 16 * f, 16)] for f in range(F4)]
    v1 = [w_v[1, pl.ds(cb + 16 * f, 16)] for f in range(F4)]
    d1 = [w_v[2, pl.ds(cb + 16 * f, 16)] for f in range(F4)]

    _zero_init(s, msg0_v, aggr_sh)
    plsc.subcore_barrier()

    def compute_chunk(ch, mv):
        chv = jnp.full((16,), ch, jnp.int32)
        for j in range(K // 16):
            idx16 = src_v[ch, pl.ds(j * 16, 16)]
            xs_v[pl.ds(j * 16, 16)] = plsc.load_gather(x_v, [idx16])

        def edge(jj):
            jv = jnp.full((16,), jj, jnp.int32)
            xj = plsc.load_gather(xs_v, [jv])
            tj = plsc.load_gather(ea_v, [chv, jv])
            for f in range(F4):
                mv[jj, pl.ds(16 * f, 16)] = jnp.maximum(
                    xj * wn[f] + tj * v1[f] + d1[f], 0.0)
        plsc.parallel_loop(0, K, unroll=4)(edge)

    bufs = ((msg0_v, sem0), (msg1_v, sem1))

    def stage(t):
        pltpu.sync_copy(src_hbm.at[s, t], src_v)
        pltpu.sync_copy(dst_hbm.at[s, t], dst_v)
        pltpu.sync_copy(ea_hbm.at[s, t], ea_v)
        for ch in range(2):
            mv, sem = bufs[ch]
            compute_chunk(ch, mv)
            pltpu.async_copy(mv, aggr_sh.at[dst_v.at[ch]], sem, add=True)

        def body(chb):
            for b in range(2):
                ch = chb + b
                mv, sem = bufs[b]
                pltpu.make_async_copy(
                    mv, aggr_sh.at[dst_v.at[ch]], sem).wait()
                compute_chunk(ch, mv)
                pltpu.async_copy(mv, aggr_sh.at[dst_v.at[ch]], sem, add=True)
        pl.loop(2, SCH, step=2)(body)
        for b in range(2):
            mv, sem = bufs[b]
            pltpu.make_async_copy(mv, aggr_sh.at[dst_v.at[b]], sem).wait()
    pl.loop(0, NST)(stage)

    plsc.subcore_barrier()
    _copy_out(c, s, aggr_sh, out0, out1)


def _sc_layer1(x, src3, dst3, ea3, p):
    kern = pl.kernel(
        _sc1_body,
        out_type=[pltpu.HBM((AGR, HH), jnp.float32),
                  pltpu.HBM((AGR, HH), jnp.float32)],
        mesh=_SC_MESH,
        compiler_params=_SC_PARAMS,
        scratch_types=[
            pltpu.VMEM((N,), jnp.float32),
            pltpu.VMEM((SCH, K), jnp.int32),
            pltpu.VMEM((SCH, K), jnp.int32),
            pltpu.VMEM((SCH, K), jnp.float32),
            pltpu.VMEM((K,), jnp.float32),
            pltpu.VMEM((K, HH), jnp.float32),
            pltpu.VMEM((K, HH), jnp.float32),
            pltpu.VMEM((8, H), jnp.float32),
            pltpu.VMEM_SHARED((AGR, HH), jnp.float32),
            pltpu.SemaphoreType.DMA,
            pltpu.SemaphoreType.DMA,
        ],
    )
    return kern(x, src3, dst3, ea3, p)


def _sc2_body(ha_hbm, hb_hbm, src_hbm, dst_hbm, ea_hbm, p_hbm,
              out0, out1,
              src_v, dst_v, ea_v, rows0_v, rows1_v, msg0_v, msg1_v, w_v,
              aggr_sh, gsem0, gsem1, ssem0, ssem1):
    c = lax.axis_index("c")
    s = lax.axis_index("s")
    pltpu.sync_copy(p_hbm, w_v)

    cb = c * HH
    v2 = [w_v[4, pl.ds(cb + 16 * f, 16)] for f in range(F4)]
    d2 = [w_v[5, pl.ds(cb + 16 * f, 16)] for f in range(F4)]

    _zero_init(s, msg0_v, aggr_sh)
    plsc.subcore_barrier()

    def issue_gather(ch, rv, gsem):
        @pl.when(c == 0)
        def _():
            pltpu.async_copy(ha_hbm.at[src_v.at[ch]], rv, gsem)

        @pl.when(c == 1)
        def _():
            pltpu.async_copy(hb_hbm.at[src_v.at[ch]], rv, gsem)

    def wait_gather(ch, rv, gsem):
        @pl.when(c == 0)
        def _():
            pltpu.make_async_copy(ha_hbm.at[src_v.at[ch]], rv, gsem).wait()

        @pl.when(c == 1)
        def _():
            pltpu.make_async_copy(hb_hbm.at[src_v.at[ch]], rv, gsem).wait()

    def compute_chunk(ch, rv, mv):
        chv = jnp.full((16,), ch, jnp.int32)

        def edge(jj):
            jv = jnp.full((16,), jj, jnp.int32)
            tj = plsc.load_gather(ea_v, [chv, jv])
            for f in range(F4):
                sl = pl.ds(16 * f, 16)
                mv[jj, sl] = jnp.maximum(
                    rv[jj, sl] + tj * v2[f] + d2[f], 0.0)
        plsc.parallel_loop(0, K, unroll=4)(edge)

    bufs = ((rows0_v, msg0_v, gsem0, ssem0), (rows1_v, msg1_v, gsem1, ssem1))

    def stage(t):
        pltpu.sync_copy(src_hbm.at[s, t], src_v)
        pltpu.sync_copy(dst_hbm.at[s, t], dst_v)
        pltpu.sync_copy(ea_hbm.at[s, t], ea_v)
        issue_gather(0, rows0_v, gsem0)
        issue_gather(1, rows1_v, gsem1)
        for ch in range(2):
            rv, mv, gsem, ssem = bufs[ch]
            wait_gather(ch, rv, gsem)
            compute_chunk(ch, rv, mv)
            issue_gather(ch + 2, rv, gsem)
            pltpu.async_copy(mv, aggr_sh.at[dst_v.at[ch]], ssem, add=True)

        def body(chb):
            for b in range(2):
                ch = chb + b
                rv, mv, gsem, ssem = bufs[b]
                wait_gather(ch, rv, gsem)
                pltpu.make_async_copy(
                    mv, aggr_sh.at[dst_v.at[ch]], ssem).wait()
                compute_chunk(ch, rv, mv)

                @pl.when(ch + 2 < SCH)
                def _():
                    issue_gather(ch + 2, rv, gsem)
                pltpu.async_copy(mv, aggr_sh.at[dst_v.at[ch]], ssem, add=True)
        pl.loop(2, SCH, step=2)(body)
        for b in range(2):
            rv, mv, gsem, ssem = bufs[b]
            pltpu.make_async_copy(mv, aggr_sh.at[dst_v.at[b]], ssem).wait()
    pl.loop(0, NST)(stage)

    plsc.subcore_barrier()
    _copy_out(c, s, aggr_sh, out0, out1)


def _sc_layer2(h1a, h1b, src3, dst3, ea3, p):
    kern = pl.kernel(
        _sc2_body,
        out_type=[pltpu.HBM((AGR, HH), jnp.float32),
                  pltpu.HBM((AGR, HH), jnp.float32)],
        mesh=_SC_MESH,
        compiler_params=_SC_PARAMS,
        scratch_types=[
            pltpu.VMEM((SCH, K), jnp.int32),
            pltpu.VMEM((SCH, K), jnp.int32),
            pltpu.VMEM((SCH, K), jnp.float32),
            pltpu.VMEM((K, HH), jnp.float32),
            pltpu.VMEM((K, HH), jnp.float32),
            pltpu.VMEM((K, HH), jnp.float32),
            pltpu.VMEM((K, HH), jnp.float32),
            pltpu.VMEM((8, H), jnp.float32),
            pltpu.VMEM_SHARED((AGR, HH), jnp.float32),
            pltpu.SemaphoreType.DMA,
            pltpu.SemaphoreType.DMA,
            pltpu.SemaphoreType.DMA,
            pltpu.SemaphoreType.DMA,
        ],
    )
    return kern(h1a, h1b, src3, dst3, ea3, p)


def _mlp_body(x2, p0, p1, pw, wa, ba, wb, bb, outa, outb):
    h0 = x2[...] * pw[0:1, :] + pw[3:4, :]
    z = h0 + jnp.concatenate([p0[...], p1[...]], axis=1)
    t = jnp.maximum(
        jnp.dot(z, wa[...], preferred_element_type=jnp.float32) + ba[...], 0.0)
    h1 = jnp.maximum(
        jnp.dot(t, wb[...], preferred_element_type=jnp.float32) + bb[...], 0.0)
    outa[...] = h1[:, :HH]
    outb[...] = h1[:, HH:]


def _mlp(x2, p0, p1, pw, wa, ba, wb, bb):
    full = lambda i: (0, 0)
    return pl.pallas_call(
        _mlp_body,
        grid=(NB,),
        in_specs=[
            pl.BlockSpec((BR, 1), lambda i: (i, 0)),
            pl.BlockSpec((BR, HH), lambda i: (i, 0)),
            pl.BlockSpec((BR, HH), lambda i: (i, 0)),
            pl.BlockSpec((8, H), full),
            pl.BlockSpec((H, H), full),
            pl.BlockSpec((1, H), full),
            pl.BlockSpec((H, H), full),
            pl.BlockSpec((1, H), full),
        ],
        out_specs=[pl.BlockSpec((BR, HH), lambda i: (i, 0)),
                   pl.BlockSpec((BR, HH), lambda i: (i, 0))],
        out_shape=[jax.ShapeDtypeStruct((N, HH), jnp.float32),
                   jax.ShapeDtypeStruct((N, HH), jnp.float32)],
    )(x2, p0, p1, pw, wa, ba, wb, bb)


def _mlp_pool_body(ha, hb, q0, q1, batch, wa, ba, wb, bb, out, sums, cnts):
    i = pl.program_id(0)

    @pl.when(i == 0)
    def _():
        sums[...] = jnp.zeros((G, H), jnp.float32)
        cnts[...] = jnp.zeros((G, H), jnp.float32)

    z = jnp.concatenate([ha[...] + q0[...], hb[...] + q1[...]], axis=1)
    t = jnp.maximum(
        jnp.dot(z, wa[...], preferred_element_type=jnp.float32) + ba[...], 0.0)
    h2 = jnp.maximum(
        jnp.dot(t, wb[...], preferred_element_type=jnp.float32) + bb[...], 0.0)
    brow = batch[0, 0, :]
    oh = (lax.broadcasted_iota(jnp.int32, (G, BR), 0)
          == brow[None, :]).astype(jnp.float32)
    sums[...] += jnp.dot(oh, h2, preferred_element_type=jnp.float32)
    cnts[...] += jnp.dot(oh, jnp.ones((BR, H), jnp.float32),
                         preferred_element_type=jnp.float32)

    @pl.when(i == NB - 1)
    def _():
        out[...] = sums[...] / jnp.maximum(cnts[...], 1.0)


def _mlp_pool(ha, hb, q0, q1, batch3, wa, ba, wb, bb):
    full = lambda i: (0, 0)
    return pl.pallas_call(
        _mlp_pool_body,
        grid=(NB,),
        in_specs=[
            pl.BlockSpec((BR, HH), lambda i: (i, 0)),
            pl.BlockSpec((BR, HH), lambda i: (i, 0)),
            pl.BlockSpec((BR, HH), lambda i: (i, 0)),
            pl.BlockSpec((BR, HH), lambda i: (i, 0)),
            pl.BlockSpec((1, 1, BR), lambda i: (i, 0, 0)),
            pl.BlockSpec((H, H), full),
            pl.BlockSpec((1, H), full),
            pl.BlockSpec((H, H), full),
            pl.BlockSpec((1, H), full),
        ],
        out_specs=pl.BlockSpec((G, H), full),
        out_shape=jax.ShapeDtypeStruct((G, H), jnp.float32),
        scratch_shapes=[
            pltpu.VMEM((G, H), jnp.float32),
            pltpu.VMEM((G, H), jnp.float32),
        ],
    )(ha, hb, q0, q1, batch3, wa, ba, wb, bb)


def kernel(x, edge_index, edge_attr, batch,
           W_node, b_node, W_edge, b_edge,
           W_e1, b_e1, W_m1a, b_m1a, W_m1b, b_m1b,
           W_e2, b_e2, W_m2a, b_m2a, W_m2b, b_m2b):
    x = x.astype(jnp.float32)
    src3 = edge_index[0].astype(jnp.int32).reshape(NS, NST, SCH, K)
    dst3 = edge_index[1].astype(jnp.int32).reshape(NS, NST, SCH, K)
    ea3 = edge_attr.astype(jnp.float32).reshape(NS, NST, SCH, K)
    batch3 = batch.astype(jnp.int32).reshape(NB, 1, BR)

    r = lambda b: b.reshape(1, H)
    p = _fold(W_node, r(b_node), W_edge, r(b_edge),
              W_e1, r(b_e1), W_e2, r(b_e2))
    p0, p1 = _sc_layer1(x, src3, dst3, ea3, p)
    h1a, h1b = _mlp(x.reshape(N, 1), p0, p1, p,
                    W_m1a, r(b_m1a), W_m1b, r(b_m1b))
    q0, q1 = _sc_layer2(h1a, h1b, src3, dst3, ea3, p)
    return _mlp_pool(h1a, h1b, q0, q1, batch3,
                     W_m2a, r(b_m2a), W_m2b, r(b_m2b))


# speedup vs baseline: 1.7330x; 1.0030x over previous
import dataclasses

import jax
import jax.numpy as jnp
from jax import lax
from jax.experimental import pallas as pl
from jax.experimental.pallas import tpu as pltpu
from jax.experimental.pallas import tpu_sc as plsc

N = 10000
E = 320000
H = 128
HH = H // 2
G = 64
NS = 16
EPT = E // NS
K = 80
NCH = EPT // K
NST = 5
SCH = NCH // NST
AGR = 10240
RPT = AGR // NS
ZB = RPT // K
NB = 5
BR = N // NB
F4 = HH // 16


def _fold_body(wn, bn, we, be, we1, be1, we2, be2, p_ref):
    v1 = jnp.dot(we[...], we1[...], preferred_element_type=jnp.float32)
    c1 = jnp.dot(be[...], we1[...], preferred_element_type=jnp.float32) + be1[...]
    v2 = jnp.dot(we[...], we2[...], preferred_element_type=jnp.float32)
    c2 = jnp.dot(be[...], we2[...], preferred_element_type=jnp.float32) + be2[...]
    z = jnp.zeros((2, H), jnp.float32)
    p_ref[...] = jnp.concatenate(
        [wn[...], v1, c1 + bn[...], bn[...], v2, c2, z], axis=0)


def _fold(wn, bn, we, be, we1, be1, we2, be2):
    return pl.pallas_call(
        _fold_body,
        out_shape=jax.ShapeDtypeStruct((8, H), jnp.float32),
    )(wn, bn, we, be, we1, be1, we2, be2)


_SC_MESH = plsc.VectorSubcoreMesh(core_axis_name="c", subcore_axis_name="s")
_SC_PARAMS = pltpu.CompilerParams()
if "needs_layout_passes" in pltpu.CompilerParams.__dataclass_fields__:
    _SC_PARAMS = dataclasses.replace(_SC_PARAMS, needs_layout_passes=False)
if "use_tc_tiling_on_sc" in pltpu.CompilerParams.__dataclass_fields__:
    _SC_PARAMS = dataclasses.replace(_SC_PARAMS, use_tc_tiling_on_sc=False)


def _zero_init(s, msg_v, aggr_sh):
    def zrow(r):
        for f in range(F4):
            msg_v[r, pl.ds(16 * f, 16)] = jnp.zeros((16,), jnp.float32)
    pl.loop(0, K)(zrow)

    def blk(i):
        pltpu.sync_copy(msg_v, aggr_sh.at[pl.ds(s * RPT + i * K, K), :])
    pl.loop(0, ZB)(blk)


def _copy_out(c, s, aggr_sh, out0, out1):
    @pl.when(c == 0)
    def _():
        pltpu.sync_copy(aggr_sh.at[pl.ds(s * RPT, RPT), :],
                        out0.at[pl.ds(s * RPT, RPT), :])

    @pl.when(c == 1)
    def _():
        pltpu.sync_copy(aggr_sh.at[pl.ds(s * RPT, RPT), :],
                        out1.at[pl.ds(s * RPT, RPT), :])


def _sc1_body(x_hbm, src_hbm, dst_hbm, ea_hbm, p_hbm,
              out0, out1,
              x_v, src_v, dst_v, ea_v, xs_v, msg0_v, msg1_v, w_v,
              aggr_sh, sem0, sem1):
    c = lax.axis_index("c")
    s = lax.axis_index("s")
    pltpu.sync_copy(x_hbm, x_v)
    pltpu.sync_copy(p_hbm, w_v)

    cb = c * HH
    wn = [w_v[0, pl.ds(cb + 16 * f, 16)] for f in range(F4)]
    v1 = [w_v[1, pl.ds(cb + 16 * f, 16)] for f in range(F4)]
    d1 = [w_v[2, pl.ds(cb + 16 * f, 16)] for f in range(F4)]

    _zero_init(s, msg0_v, aggr_sh)
    plsc.subcore_barrier()

    def compute_chunk(ch, mv):
        chv = jnp.full((16,), ch, jnp.int32)
        for j in range(K // 16):
            idx16 = src_v[ch, pl.ds(j * 16, 16)]
            xs_v[pl.ds(j * 16, 16)] = plsc.load_gather(x_v, [idx16])

        def edge(jj):
            jv = jnp.full((16,), jj, jnp.int32)
            xj = plsc.load_gather(xs_v, [jv])
            tj = plsc.load_gather(ea_v, [chv, jv])
            for f in range(F4):
                mv[jj, pl.ds(16 * f, 16)] = jnp.maximum(
                    xj * wn[f] + tj * v1[f] + d1[f], 0.0)
        plsc.parallel_loop(0, K, unroll=2)(edge)

    bufs = ((msg0_v, sem0), (msg1_v, sem1))

    def stage(t):
        pltpu.sync_copy(src_hbm.at[s, t], src_v)
        pltpu.sync_copy(dst_hbm.at[s, t], dst_v)
        pltpu.sync_copy(ea_hbm.at[s, t], ea_v)
        for ch in range(2):
            mv, sem = bufs[ch]
            compute_chunk(ch, mv)
            pltpu.async_copy(mv, aggr_sh.at[dst_v.at[ch]], sem, add=True)

        def body(chb):
            for b in range(2):
                ch = chb + b
                mv, sem = bufs[b]
                pltpu.make_async_copy(
                    mv, aggr_sh.at[dst_v.at[ch]], sem).wait()
                compute_chunk(ch, mv)
                pltpu.async_copy(mv, aggr_sh.at[dst_v.at[ch]], sem, add=True)
        pl.loop(2, SCH, step=2)(body)
        for b in range(2):
            mv, sem = bufs[b]
            pltpu.make_async_copy(mv, aggr_sh.at[dst_v.at[b]], sem).wait()
    pl.loop(0, NST)(stage)

    plsc.subcore_barrier()
    _copy_out(c, s, aggr_sh, out0, out1)


def _sc_layer1(x, src3, dst3, ea3, p):
    kern = pl.kernel(
        _sc1_body,
        out_type=[pltpu.HBM((AGR, HH), jnp.float32),
                  pltpu.HBM((AGR, HH), jnp.float32)],
        mesh=_SC_MESH,
        compiler_params=_SC_PARAMS,
        scratch_types=[
            pltpu.VMEM((N,), jnp.float32),
            pltpu.VMEM((SCH, K), jnp.int32),
            pltpu.VMEM((SCH, K), jnp.int32),
            pltpu.VMEM((SCH, K), jnp.float32),
            pltpu.VMEM((K,), jnp.float32),
            pltpu.VMEM((K, HH), jnp.float32),
            pltpu.VMEM((K, HH), jnp.float32),
            pltpu.VMEM((8, H), jnp.float32),
            pltpu.VMEM_SHARED((AGR, HH), jnp.float32),
            pltpu.SemaphoreType.DMA,
            pltpu.SemaphoreType.DMA,
        ],
    )
    return kern(x, src3, dst3, ea3, p)


def _sc2_body(ha_hbm, hb_hbm, src_hbm, dst_hbm, ea_hbm, p_hbm,
              out0, out1,
              src_v, dst_v, ea_v, rows0_v, rows1_v, msg0_v, msg1_v, w_v,
              aggr_sh, gsem0, gsem1, ssem0, ssem1):
    c = lax.axis_index("c")
    s = lax.axis_index("s")
    pltpu.sync_copy(p_hbm, w_v)

    cb = c * HH
    v2 = [w_v[4, pl.ds(cb + 16 * f, 16)] for f in range(F4)]
    d2 = [w_v[5, pl.ds(cb + 16 * f, 16)] for f in range(F4)]

    _zero_init(s, msg0_v, aggr_sh)
    plsc.subcore_barrier()

    def issue_gather(ch, rv, gsem):
        @pl.when(c == 0)
        def _():
            pltpu.async_copy(ha_hbm.at[src_v.at[ch]], rv, gsem)

        @pl.when(c == 1)
        def _():
            pltpu.async_copy(hb_hbm.at[src_v.at[ch]], rv, gsem)

    def wait_gather(ch, rv, gsem):
        @pl.when(c == 0)
        def _():
            pltpu.make_async_copy(ha_hbm.at[src_v.at[ch]], rv, gsem).wait()

        @pl.when(c == 1)
        def _():
            pltpu.make_async_copy(hb_hbm.at[src_v.at[ch]], rv, gsem).wait()

    def compute_chunk(ch, rv, mv):
        chv = jnp.full((16,), ch, jnp.int32)

        def edge(jj):
            jv = jnp.full((16,), jj, jnp.int32)
            tj = plsc.load_gather(ea_v, [chv, jv])
            for f in range(F4):
                sl = pl.ds(16 * f, 16)
                mv[jj, sl] = jnp.maximum(
                    rv[jj, sl] + tj * v2[f] + d2[f], 0.0)
        plsc.parallel_loop(0, K, unroll=2)(edge)

    bufs = ((rows0_v, msg0_v, gsem0, ssem0), (rows1_v, msg1_v, gsem1, ssem1))

    def stage(t):
        pltpu.sync_copy(src_hbm.at[s, t], src_v)
        pltpu.sync_copy(dst_hbm.at[s, t], dst_v)
        pltpu.sync_copy(ea_hbm.at[s, t], ea_v)
        issue_gather(0, rows0_v, gsem0)
        issue_gather(1, rows1_v, gsem1)
        for ch in range(2):
            rv, mv, gsem, ssem = bufs[ch]
            wait_gather(ch, rv, gsem)
            compute_chunk(ch, rv, mv)
            issue_gather(ch + 2, rv, gsem)
            pltpu.async_copy(mv, aggr_sh.at[dst_v.at[ch]], ssem, add=True)

        def body(chb):
            for b in range(2):
                ch = chb + b
                rv, mv, gsem, ssem = bufs[b]
                wait_gather(ch, rv, gsem)
                pltpu.make_async_copy(
                    mv, aggr_sh.at[dst_v.at[ch]], ssem).wait()
                compute_chunk(ch, rv, mv)

                @pl.when(ch + 2 < SCH)
                def _():
                    issue_gather(ch + 2, rv, gsem)
                pltpu.async_copy(mv, aggr_sh.at[dst_v.at[ch]], ssem, add=True)
        pl.loop(2, SCH, step=2)(body)
        for b in range(2):
            rv, mv, gsem, ssem = bufs[b]
            pltpu.make_async_copy(mv, aggr_sh.at[dst_v.at[b]], ssem).wait()
    pl.loop(0, NST)(stage)

    plsc.subcore_barrier()
    _copy_out(c, s, aggr_sh, out0, out1)


def _sc_layer2(h1a, h1b, src3, dst3, ea3, p):
    kern = pl.kernel(
        _sc2_body,
        out_type=[pltpu.HBM((AGR, HH), jnp.float32),
                  pltpu.HBM((AGR, HH), jnp.float32)],
        mesh=_SC_MESH,
        compiler_params=_SC_PARAMS,
        scratch_types=[
            pltpu.VMEM((SCH, K), jnp.int32),
            pltpu.VMEM((SCH, K), jnp.int32),
            pltpu.VMEM((SCH, K), jnp.float32),
            pltpu.VMEM((K, HH), jnp.float32),
            pltpu.VMEM((K, HH), jnp.float32),
            pltpu.VMEM((K, HH), jnp.float32),
            pltpu.VMEM((K, HH), jnp.float32),
            pltpu.VMEM((8, H), jnp.float32),
            pltpu.VMEM_SHARED((AGR, HH), jnp.float32),
            pltpu.SemaphoreType.DMA,
            pltpu.SemaphoreType.DMA,
            pltpu.SemaphoreType.DMA,
            pltpu.SemaphoreType.DMA,
        ],
    )
    return kern(h1a, h1b, src3, dst3, ea3, p)


def _mlp_body(x2, p0, p1, pw, wa, ba, wb, bb, outa, outb):
    h0 = x2[...] * pw[0:1, :] + pw[3:4, :]
    z = h0 + jnp.concatenate([p0[...], p1[...]], axis=1)
    t = jnp.maximum(
        jnp.dot(z, wa[...], preferred_element_type=jnp.float32) + ba[...], 0.0)
    h1 = jnp.maximum(
        jnp.dot(t, wb[...], preferred_element_type=jnp.float32) + bb[...], 0.0)
    outa[...] = h1[:, :HH]
    outb[...] = h1[:, HH:]


def _mlp(x2, p0, p1, pw, wa, ba, wb, bb):
    full = lambda i: (0, 0)
    return pl.pallas_call(
        _mlp_body,
        grid=(NB,),
        in_specs=[
            pl.BlockSpec((BR, 1), lambda i: (i, 0)),
            pl.BlockSpec((BR, HH), lambda i: (i, 0)),
            pl.BlockSpec((BR, HH), lambda i: (i, 0)),
            pl.BlockSpec((8, H), full),
            pl.BlockSpec((H, H), full),
            pl.BlockSpec((1, H), full),
            pl.BlockSpec((H, H), full),
            pl.BlockSpec((1, H), full),
        ],
        out_specs=[pl.BlockSpec((BR, HH), lambda i: (i, 0)),
                   pl.BlockSpec((BR, HH), lambda i: (i, 0))],
        out_shape=[jax.ShapeDtypeStruct((N, HH), jnp.float32),
                   jax.ShapeDtypeStruct((N, HH), jnp.float32)],
    )(x2, p0, p1, pw, wa, ba, wb, bb)


def _mlp_pool_body(ha, hb, q0, q1, batch, wa, ba, wb, bb, out, sums, cnts):
    i = pl.program_id(0)

    @pl.when(i == 0)
    def _():
        sums[...] = jnp.zeros((G, H), jnp.float32)
        cnts[...] = jnp.zeros((G, H), jnp.float32)

    z = jnp.concatenate([ha[...] + q0[...], hb[...] + q1[...]], axis=1)
    t = jnp.maximum(
        jnp.dot(z, wa[...], preferred_element_type=jnp.float32) + ba[...], 0.0)
    h2 = jnp.maximum(
        jnp.dot(t, wb[...], preferred_element_type=jnp.float32) + bb[...], 0.0)
    brow = batch[0, 0, :]
    oh = (lax.broadcasted_iota(jnp.int32, (G, BR), 0)
          == brow[None, :]).astype(jnp.float32)
    sums[...] += jnp.dot(oh, h2, preferred_element_type=jnp.float32)
    cnts[...] += jnp.dot(oh, jnp.ones((BR, H), jnp.float32),
                         preferred_element_type=jnp.float32)

    @pl.when(i == NB - 1)
    def _():
        out[...] = sums[...] / jnp.maximum(cnts[...], 1.0)


def _mlp_pool(ha, hb, q0, q1, batch3, wa, ba, wb, bb):
    full = lambda i: (0, 0)
    return pl.pallas_call(
        _mlp_pool_body,
        grid=(NB,),
        in_specs=[
            pl.BlockSpec((BR, HH), lambda i: (i, 0)),
            pl.BlockSpec((BR, HH), lambda i: (i, 0)),
            pl.BlockSpec((BR, HH), lambda i: (i, 0)),
            pl.BlockSpec((BR, HH), lambda i: (i, 0)),
            pl.BlockSpec((1, 1, BR), lambda i: (i, 0, 0)),
            pl.BlockSpec((H, H), full),
            pl.BlockSpec((1, H), full),
            pl.BlockSpec((H, H), full),
            pl.BlockSpec((1, H), full),
        ],
        out_specs=pl.BlockSpec((G, H), full),
        out_shape=jax.ShapeDtypeStruct((G, H), jnp.float32),
        scratch_shapes=[
            pltpu.VMEM((G, H), jnp.float32),
            pltpu.VMEM((G, H), jnp.float32),
        ],
    )(ha, hb, q0, q1, batch3, wa, ba, wb, bb)


def kernel(x, edge_index, edge_attr, batch,
           W_node, b_node, W_edge, b_edge,
           W_e1, b_e1, W_m1a, b_m1a, W_m1b, b_m1b,
           W_e2, b_e2, W_m2a, b_m2a, W_m2b, b_m2b):
    x = x.astype(jnp.float32)
    src3 = edge_index[0].astype(jnp.int32).reshape(NS, NST, SCH, K)
    dst3 = edge_index[1].astype(jnp.int32).reshape(NS, NST, SCH, K)
    ea3 = edge_attr.astype(jnp.float32).reshape(NS, NST, SCH, K)
    batch3 = batch.astype(jnp.int32).reshape(NB, 1, BR)

    r = lambda b: b.reshape(1, H)
    p = _fold(W_node, r(b_node), W_edge, r(b_edge),
              W_e1, r(b_e1), W_e2, r(b_e2))
    p0, p1 = _sc_layer1(x, src3, dst3, ea3, p)
    h1a, h1b = _mlp(x.reshape(N, 1), p0, p1, p,
                    W_m1a, r(b_m1a), W_m1b, r(b_m1b))
    q0, q1 = _sc_layer2(h1a, h1b, src3, dst3, ea3, p)
    return _mlp_pool(h1a, h1b, q0, q1, batch3,
                     W_m2a, r(b_m2a), W_m2b, r(b_m2b))
